# baseline (speedup 1.0000x reference)
import functools

import jax
import jax.numpy as jnp
from jax.experimental import pallas as pl
from jax.experimental.pallas import tpu as pltpu


def _moe_body(x_ref, gw_ref, gb_ref, w_ref, b_ref, out_ref, w_scr, *, n_experts):
    @pl.when(pl.program_id(1) == 0)
    def _cast_slab():
        w_scr[...] = w_ref[...].astype(jnp.bfloat16)

    xb = x_ref[...].astype(jnp.bfloat16)
    logits = jnp.dot(xb, gw_ref[...].astype(jnp.bfloat16), preferred_element_type=jnp.float32)
    logits = logits + gb_ref[...]
    m = jnp.max(logits, axis=-1, keepdims=True)
    p = jnp.exp(logits - m)
    g = p / jnp.sum(p, axis=-1, keepdims=True)

    acc = jnp.zeros(out_ref.shape, jnp.float32)
    for e in range(n_experts):
        ye = jnp.dot(xb, w_scr[e], preferred_element_type=jnp.float32)
        acc = acc + g[:, e : e + 1] * (ye + b_ref[e][None, :])
    out_ref[...] = acc


def kernel(x, gate_W, gate_b, expert_W, expert_b):
    n_tok, k = x.shape
    n_exp, _, n_out = expert_W.shape

    bm = min(512, n_tok)
    bn = min(256, n_out)
    grid = (n_out // bn, n_tok // bm)

    gb2 = gate_b.reshape(1, n_exp)

    body = functools.partial(_moe_body, n_experts=n_exp)
    return pl.pallas_call(
        body,
        grid=grid,
        in_specs=[
            pl.BlockSpec((bm, k), lambda n, m: (m, 0)),
            pl.BlockSpec((k, n_exp), lambda n, m: (0, 0)),
            pl.BlockSpec((1, n_exp), lambda n, m: (0, 0)),
            pl.BlockSpec((n_exp, k, bn), lambda n, m: (0, 0, n)),
            pl.BlockSpec((n_exp, bn), lambda n, m: (0, n)),
        ],
        out_specs=pl.BlockSpec((bm, bn), lambda n, m: (m, n)),
        out_shape=jax.ShapeDtypeStruct((n_tok, n_out), jnp.float32),
        scratch_shapes=[pltpu.VMEM((n_exp, k, bn), jnp.bfloat16)],
        compiler_params=pltpu.CompilerParams(
            dimension_semantics=("arbitrary", "arbitrary"),
        ),
    )(x, gate_W, gb2, expert_W, expert_b)
